# baseline (speedup 1.0000x reference)
import functools

import jax
import jax.numpy as jnp
from jax import lax
from jax.experimental import pallas as pl
from jax.experimental.pallas import tpu as pltpu
from jax.experimental.pallas import tpu_sc as plsc

N = 10000
E = 320000
D = 128
HID = 128
OUT = 128
EF = 16

NC = 2
NS = 16
NW = NC * NS
EPW = E // NW
CH = 80
NCHUNK = EPW // CH
HZR = 1000


def _silu(x):
    return x * jax.nn.sigmoid(x)


def _pre_body(nf_ref, w1a_ref, w1b_ref, be1_ref, a_ref, b_ref):
    x = nf_ref[...]
    a_ref[...] = (
        jnp.dot(x, w1a_ref[...], preferred_element_type=jnp.float32)
        + be1_ref[...]
    )
    b_ref[...] = jnp.dot(x, w1b_ref[...], preferred_element_type=jnp.float32)


def _node_precompute(node_feat, w1a, w1b, be1):
    tile = 2000
    grid = N // tile
    return pl.pallas_call(
        _pre_body,
        grid=(grid,),
        in_specs=[
            pl.BlockSpec((tile, D), lambda i: (i, 0)),
            pl.BlockSpec((D, HID), lambda i: (0, 0)),
            pl.BlockSpec((D, HID), lambda i: (0, 0)),
            pl.BlockSpec((1, HID), lambda i: (0, 0)),
        ],
        out_specs=[
            pl.BlockSpec((tile, HID), lambda i: (i, 0)),
            pl.BlockSpec((tile, HID), lambda i: (i, 0)),
        ],
        out_shape=[
            jax.ShapeDtypeStruct((N, HID), jnp.float32),
            jax.ShapeDtypeStruct((N, HID), jnp.float32),
        ],
    )(node_feat, w1a, w1b, be1)


_MESH = plsc.VectorSubcoreMesh(core_axis_name="c", subcore_axis_name="s")


@functools.partial(
    pl.kernel,
    out_type=(
        jax.ShapeDtypeStruct((E, HID), jnp.float32),
        jax.ShapeDtypeStruct((E, 16), jnp.float32),
    ),
    mesh=_MESH,
    scratch_types=[
        pltpu.VMEM((NCHUNK, CH), jnp.int32),
        pltpu.VMEM((NCHUNK, CH), jnp.int32),
        pltpu.VMEM((CH, HID), jnp.float32),
        pltpu.VMEM((CH, HID), jnp.float32),
        pltpu.VMEM((CH, HID), jnp.float32),
        pltpu.VMEM((CH, HID), jnp.float32),
        pltpu.VMEM((N,), jnp.float32),
        pltpu.VMEM((N,), jnp.float32),
        pltpu.VMEM((N,), jnp.float32),
        pltpu.VMEM((CH, 16), jnp.float32),
        pltpu.VMEM((CH, 16), jnp.float32),
        pltpu.SemaphoreType.DMA,
        pltpu.SemaphoreType.DMA,
        pltpu.SemaphoreType.DMA,
        pltpu.SemaphoreType.DMA,
    ],
    compiler_params=pltpu.CompilerParams(needs_layout_passes=False),
)
def _edge_gather(a_hbm, b_hbm, cpx_hbm, cpy_hbm, cpz_hbm, src_hbm, dst_hbm,
                 g_hbm, xd_hbm,
                 idxs_v, idxd_v, bufa0, bufa1, bufb0, bufb1, cpx, cpy, cpz,
                 bufxd0, bufxd1, gsem0, gsem1, wsem0, wsem1):
    wid = lax.axis_index("s") * NC + lax.axis_index("c")
    base = wid * EPW
    pltpu.sync_copy(src_hbm.at[wid], idxs_v)
    pltpu.sync_copy(dst_hbm.at[wid], idxd_v)
    pltpu.sync_copy(cpx_hbm, cpx)
    pltpu.sync_copy(cpy_hbm, cpy)
    pltpu.sync_copy(cpz_hbm, cpz)

    bufa = (bufa0, bufa1)
    bufb = (bufb0, bufb1)
    bufxd = (bufxd0, bufxd1)
    gsem = (gsem0, gsem1)
    wsem = (wsem0, wsem1)

    zero16 = jnp.zeros((16,), jnp.float32)

    def zrow(r, carry):
        bufxd0[r] = zero16
        bufxd1[r] = zero16
        return carry

    lax.fori_loop(0, CH, zrow, 0)

    lanes = lax.iota(jnp.int32, 16)

    def fire_gather(j, sb):
        pltpu.async_copy(a_hbm.at[idxs_v.at[j]], bufa[sb], gsem[sb])
        pltpu.async_copy(b_hbm.at[idxd_v.at[j]], bufb[sb], gsem[sb])

    def wait_gather(j, sb):
        pltpu.make_async_copy(a_hbm.at[idxs_v.at[j]], bufa[sb],
                              gsem[sb]).wait()
        pltpu.make_async_copy(b_hbm.at[idxd_v.at[j]], bufb[sb],
                              gsem[sb]).wait()

    def fire_writes(j, sb):
        off = base + j * CH
        pltpu.async_copy(bufa[sb], g_hbm.at[pl.ds(off, CH)], wsem[sb])
        pltpu.async_copy(bufxd[sb], xd_hbm.at[pl.ds(off, CH)], wsem[sb])

    def wait_writes(sb):
        pltpu.make_async_copy(bufa[sb], g_hbm.at[pl.ds(0, CH)],
                              wsem[sb]).wait()
        pltpu.make_async_copy(bufxd[sb], xd_hbm.at[pl.ds(0, CH)],
                              wsem[sb]).wait()

    def compute(j, sb):
        bxd = bufxd[sb]
        ba = bufa[sb]
        bb = bufb[sb]

        def grp(k, carry2):
            sidx = idxs_v[j, pl.ds(k * 16, 16)]
            didx = idxd_v[j, pl.ds(k * 16, 16)]
            dx = plsc.load_gather(cpx, [sidx]) - plsc.load_gather(cpx, [didx])
            dy = plsc.load_gather(cpy, [sidx]) - plsc.load_gather(cpy, [didx])
            dz = plsc.load_gather(cpz, [sidx]) - plsc.load_gather(cpz, [didx])
            rad = dx * dx + dy * dy + dz * dz
            ridx = lanes + k * 16
            plsc.store_scatter(bxd, [ridx, jnp.zeros((16,), jnp.int32)], dx)
            plsc.store_scatter(bxd, [ridx, jnp.full((16,), 1, jnp.int32)], dy)
            plsc.store_scatter(bxd, [ridx, jnp.full((16,), 2, jnp.int32)], dz)
            plsc.store_scatter(bxd, [ridx, jnp.full((16,), 3, jnp.int32)], rad)
            return carry2

        lax.fori_loop(0, CH // 16, grp, 0)

        def row(r, carry2):
            for kk in range(HID // 16):
                sl = pl.ds(kk * 16, 16)
                ba[r, sl] = ba[r, sl] + bb[r, sl]
            return carry2

        lax.fori_loop(0, CH, row, 0)

    def do_chunk(j, sb, ob):
        @pl.when(j + 1 < NCHUNK)
        def _prefetch():
            @pl.when(j >= 1)
            def _drain():
                wait_writes(ob)

            fire_gather(j + 1, ob)

        wait_gather(j, sb)
        compute(j, sb)
        fire_writes(j, sb)

    fire_gather(0, 0)

    def pair(jj, carry):
        j0 = 2 * jj
        do_chunk(j0, 0, 1)

        @pl.when(j0 + 1 < NCHUNK)
        def _odd():
            do_chunk(j0 + 1, 1, 0)

        return carry

    lax.fori_loop(0, (NCHUNK + 1) // 2, pair, 0)
    wait_writes(0)
    wait_writes(1)


def _edge_body(g_ref, xd_ref, ef_ref, wef_ref, wr_ref, we2_ref, be2_ref,
               wc1_ref, bc1_ref, wc2_ref, msgh_ref, msgx_ref):
    g = g_ref[...]
    xd = xd_ref[...]
    radial = xd[:, 3:4]
    pre = (
        g
        + radial * wr_ref[...]
        + jnp.dot(ef_ref[...], wef_ref[...], preferred_element_type=jnp.float32)
    )
    bf16 = jnp.bfloat16
    m1 = _silu(pre)
    mh = _silu(
        jnp.dot(m1.astype(bf16), we2_ref[...].astype(bf16),
                preferred_element_type=jnp.float32)
        + be2_ref[...]
    )
    t = _silu(
        jnp.dot(mh.astype(bf16), wc1_ref[...].astype(bf16),
                preferred_element_type=jnp.float32)
        + bc1_ref[...]
    )
    coef = jnp.sum(t * wc2_ref[...], axis=1, keepdims=True)
    inv = 1.0 / (jnp.sqrt(radial) + 1e-30)
    lane = lax.broadcasted_iota(jnp.int32, (1, 16), 1)
    mask3 = jnp.where(lane < 3, 1.0, 0.0)
    deg1 = jnp.where(lane == 3, 1.0, 0.0)
    msgh_ref[...] = mh
    msgx_ref[...] = coef * inv * xd * mask3 + deg1


def _edge_mlp(g, xd, edge_feat, wef, wr, we2, be2, wc1, bc1, wc2_row):
    tile = 2000
    grid = E // tile
    return pl.pallas_call(
        _edge_body,
        grid=(grid,),
        in_specs=[
            pl.BlockSpec((tile, HID), lambda i: (i, 0)),
            pl.BlockSpec((tile, 16), lambda i: (i, 0)),
            pl.BlockSpec((tile, EF), lambda i: (i, 0)),
            pl.BlockSpec((EF, HID), lambda i: (0, 0)),
            pl.BlockSpec((1, HID), lambda i: (0, 0)),
            pl.BlockSpec((HID, HID), lambda i: (0, 0)),
            pl.BlockSpec((1, HID), lambda i: (0, 0)),
            pl.BlockSpec((HID, HID), lambda i: (0, 0)),
            pl.BlockSpec((1, HID), lambda i: (0, 0)),
            pl.BlockSpec((1, HID), lambda i: (0, 0)),
        ],
        out_specs=[
            pl.BlockSpec((tile, HID), lambda i: (i, 0)),
            pl.BlockSpec((tile, 16), lambda i: (i, 0)),
        ],
        out_shape=[
            jax.ShapeDtypeStruct((E, HID), jnp.float32),
            jax.ShapeDtypeStruct((E, 16), jnp.float32),
        ],
    )(g, xd, edge_feat, wef, wr, we2, be2, wc1, bc1, wc2_row)


HROWS = 10000


@functools.partial(
    pl.kernel,
    out_type=jax.ShapeDtypeStruct((NC, N, HID), jnp.float32),
    mesh=_MESH,
    scratch_types=[
        pltpu.VMEM((NCHUNK, CH), jnp.int32),
        pltpu.VMEM((CH, HID), jnp.float32),
        pltpu.VMEM_SHARED((HROWS, HID), jnp.float32),
        pltpu.SemaphoreType.DMA,
    ],
    compiler_params=pltpu.CompilerParams(needs_layout_passes=False),
)
def _scatter_h(msgh_hbm, dst_hbm, zh_hbm, hpart, idx_v, bufh, hacc, sem):
    cid = lax.axis_index("c")
    sid = lax.axis_index("s")
    wid = sid * NC + cid
    base = wid * EPW

    @pl.when(sid < 10)
    def _zero_main():
        pltpu.sync_copy(zh_hbm.at[pl.ds(sid * HZR, HZR)],
                        hacc.at[pl.ds(sid * HZR, HZR)])

    pltpu.sync_copy(dst_hbm.at[wid], idx_v)
    plsc.subcore_barrier()

    def chunk(j, carry):
        off = base + j * CH
        pltpu.async_copy(msgh_hbm.at[pl.ds(off, CH)], bufh, sem).wait()
        pltpu.sync_copy(bufh, hacc.at[idx_v.at[j]], add=True)
        return carry

    lax.fori_loop(0, NCHUNK, chunk, 0)
    plsc.subcore_barrier()

    @pl.when(sid < 10)
    def _copy_out():
        pltpu.sync_copy(hacc.at[pl.ds(sid * HZR, HZR)],
                        hpart.at[cid, pl.ds(sid * HZR, HZR)])


@functools.partial(
    pl.kernel,
    out_type=jax.ShapeDtypeStruct((NC, N, HID), jnp.float32),
    mesh=_MESH,
    scratch_types=[
        pltpu.VMEM((NCHUNK, CH), jnp.int32),
        pltpu.VMEM((CH, 16), jnp.float32),
        pltpu.VMEM((CH, HID), jnp.float32),
        pltpu.VMEM_SHARED((HROWS, HID), jnp.float32),
        pltpu.SemaphoreType.DMA,
    ],
    compiler_params=pltpu.CompilerParams(needs_layout_passes=False),
)
def _scatter_x(msgx_hbm, dst_hbm, zh_hbm, xpart, idx_v, bufx, bufx128, xacc,
               sem):
    cid = lax.axis_index("c")
    sid = lax.axis_index("s")
    wid = sid * NC + cid
    base = wid * EPW

    @pl.when(sid < 10)
    def _zero_main():
        pltpu.sync_copy(zh_hbm.at[pl.ds(sid * HZR, HZR)],
                        xacc.at[pl.ds(sid * HZR, HZR)])

    pltpu.sync_copy(dst_hbm.at[wid], idx_v)
    zero16 = jnp.zeros((16,), jnp.float32)

    def zrow(r, carry):
        for kk in range(1, HID // 16):
            bufx128[r, pl.ds(kk * 16, 16)] = zero16
        return carry

    lax.fori_loop(0, CH, zrow, 0)
    plsc.subcore_barrier()

    def chunk(j, carry):
        off = base + j * CH
        pltpu.async_copy(msgx_hbm.at[pl.ds(off, CH)], bufx, sem).wait()

        def row(r, carry2):
            bufx128[r, pl.ds(0, 16)] = bufx[r]
            return carry2

        lax.fori_loop(0, CH, row, 0)
        pltpu.sync_copy(bufx128, xacc.at[idx_v.at[j]], add=True)
        return carry

    lax.fori_loop(0, NCHUNK, chunk, 0)
    plsc.subcore_barrier()

    @pl.when(sid < 10)
    def _copy_out():
        pltpu.sync_copy(xacc.at[pl.ds(sid * HZR, HZR)],
                        xpart.at[cid, pl.ds(sid * HZR, HZR)])


def _node_body(nf_ref, cp_ref, h0_ref, h1_ref, x0_ref, x1_ref,
               wn1a_ref, wn1b_ref, bn1_ref, wn2_ref, bn2_ref,
               gamma_ref, beta_ref, h_ref, xp_ref):
    nf = nf_ref[...]
    hn = h0_ref[...] + h1_ref[...]
    xs = x0_ref[...][:, :16] + x1_ref[...][:, :16]
    deg = jnp.maximum(xs[:, 3:4], 1.0)
    lane = lax.broadcasted_iota(jnp.int32, (1, 16), 1)
    mask3 = jnp.where(lane < 3, 1.0, 0.0)
    h1v = _silu(
        jnp.dot(nf, wn1a_ref[...], preferred_element_type=jnp.float32)
        + jnp.dot(hn, wn1b_ref[...], preferred_element_type=jnp.float32)
        + bn1_ref[...]
    )
    h2 = (
        jnp.dot(h1v, wn2_ref[...], preferred_element_type=jnp.float32)
        + bn2_ref[...]
    )
    mean = jnp.mean(h2, axis=0, keepdims=True)
    var = jnp.mean(h2 * h2, axis=0, keepdims=True) - mean * mean
    h_ref[...] = (
        (h2 - mean) / jnp.sqrt(var + 1e-5) * gamma_ref[...] + beta_ref[...]
    )
    xp_ref[...] = cp_ref[...] + xs * mask3 / deg


def _node_update(node_feat, cp, h0, h1, x0, x1, wn1a, wn1b, bn1, wn2, bn2,
                 gamma, beta):
    return pl.pallas_call(
        _node_body,
        out_shape=[
            jax.ShapeDtypeStruct((N, OUT), jnp.float32),
            jax.ShapeDtypeStruct((N, 16), jnp.float32),
        ],
    )(node_feat, cp, h0, h1, x0, x1, wn1a, wn1b, bn1, wn2, bn2, gamma, beta)


def kernel(node_feat, coord_feat, edge_index, edge_feat, W_e1, b_e1, W_e2,
           b_e2, W_n1, b_n1, W_n2, b_n2, W_c1, b_c1, W_c2, bn_gamma, bn_beta):
    f32 = jnp.float32
    w1a = W_e1[:D]
    w1b = W_e1[D:2 * D]
    wr = W_e1[2 * D:2 * D + 1]
    wef = W_e1[2 * D + 1:]
    be1 = b_e1.reshape(1, HID)
    be2 = b_e2.reshape(1, HID)
    bc1 = b_c1.reshape(1, HID)
    bn1 = b_n1.reshape(1, HID)
    bn2 = b_n2.reshape(1, OUT)
    wc2_row = W_c2.reshape(1, HID)
    gamma = bn_gamma.reshape(1, OUT)
    beta = bn_beta.reshape(1, OUT)
    wn1a = W_n1[:D]
    wn1b = W_n1[D:]

    cp = jnp.pad(coord_feat, ((0, 0), (0, 13)))
    cpx_t = jnp.asarray(coord_feat[:, 0], f32)
    cpy_t = jnp.asarray(coord_feat[:, 1], f32)
    cpz_t = jnp.asarray(coord_feat[:, 2], f32)
    src3 = edge_index[0].reshape(NW, NCHUNK, CH)
    dst3 = edge_index[1].reshape(NW, NCHUNK, CH)

    a, b = _node_precompute(node_feat, w1a, w1b, be1)
    g, xd = _edge_gather(a, b, cpx_t, cpy_t, cpz_t, src3, dst3)
    msgh, msgx = _edge_mlp(g, xd, edge_feat, wef, wr, W_e2, be2, W_c1, bc1,
                           wc2_row)
    zh = jnp.zeros((HROWS, HID), f32)
    hpart = _scatter_h(msgh, dst3, zh)
    xpart = _scatter_x(msgx, dst3, zh)
    h, xp = _node_update(node_feat, cp, hpart[0], hpart[1], xpart[0],
                         xpart[1], wn1a, wn1b, bn1, W_n2, bn2, gamma, beta)
    return (h, xp[:, :3])


# speedup vs baseline: 1.2864x; 1.2864x over previous
import functools

import jax
import jax.numpy as jnp
from jax import lax
from jax.experimental import pallas as pl
from jax.experimental.pallas import tpu as pltpu
from jax.experimental.pallas import tpu_sc as plsc

N = 10000
E = 320000
D = 128
HID = 128
OUT = 128
EF = 16

NC = 2
NS = 16
NW = NC * NS
EPW = E // NW
CH = 80
NCHUNK = EPW // CH
HZR = 1000


def _silu(x):
    return x * jax.nn.sigmoid(x)


def _pre_body(nf_ref, w1a_ref, w1b_ref, be1_ref, a_ref, b_ref):
    x = nf_ref[...]
    a_ref[...] = (
        jnp.dot(x, w1a_ref[...], preferred_element_type=jnp.float32)
        + be1_ref[...]
    )
    b_ref[...] = jnp.dot(x, w1b_ref[...], preferred_element_type=jnp.float32)


def _node_precompute(node_feat, w1a, w1b, be1):
    tile = 2000
    grid = N // tile
    return pl.pallas_call(
        _pre_body,
        grid=(grid,),
        in_specs=[
            pl.BlockSpec((tile, D), lambda i: (i, 0)),
            pl.BlockSpec((D, HID), lambda i: (0, 0)),
            pl.BlockSpec((D, HID), lambda i: (0, 0)),
            pl.BlockSpec((1, HID), lambda i: (0, 0)),
        ],
        out_specs=[
            pl.BlockSpec((tile, HID), lambda i: (i, 0)),
            pl.BlockSpec((tile, HID), lambda i: (i, 0)),
        ],
        out_shape=[
            jax.ShapeDtypeStruct((N, HID), jnp.float32),
            jax.ShapeDtypeStruct((N, HID), jnp.float32),
        ],
    )(node_feat, w1a, w1b, be1)


_MESH = plsc.VectorSubcoreMesh(core_axis_name="c", subcore_axis_name="s")


@functools.partial(
    pl.kernel,
    out_type=(
        jax.ShapeDtypeStruct((E, HID), jnp.float32),
        jax.ShapeDtypeStruct((E, 16), jnp.float32),
    ),
    mesh=_MESH,
    scratch_types=[
        pltpu.VMEM((NCHUNK, CH), jnp.int32),
        pltpu.VMEM((NCHUNK, CH), jnp.int32),
        pltpu.VMEM((CH, HID), jnp.float32),
        pltpu.VMEM((CH, HID), jnp.float32),
        pltpu.VMEM((CH, HID), jnp.float32),
        pltpu.VMEM((CH, HID), jnp.float32),
        pltpu.VMEM((N,), jnp.float32),
        pltpu.VMEM((N,), jnp.float32),
        pltpu.VMEM((N,), jnp.float32),
        pltpu.VMEM((CH, 16), jnp.float32),
        pltpu.VMEM((CH, 16), jnp.float32),
        pltpu.SemaphoreType.DMA,
        pltpu.SemaphoreType.DMA,
        pltpu.SemaphoreType.DMA,
        pltpu.SemaphoreType.DMA,
    ],
    compiler_params=pltpu.CompilerParams(needs_layout_passes=False),
)
def _edge_gather(a_hbm, b_hbm, cpx_hbm, cpy_hbm, cpz_hbm, src_hbm, dst_hbm,
                 g_hbm, xd_hbm,
                 idxs_v, idxd_v, bufa0, bufa1, bufb0, bufb1, cpx, cpy, cpz,
                 bufxd0, bufxd1, gsem0, gsem1, wsem0, wsem1):
    wid = lax.axis_index("s") * NC + lax.axis_index("c")
    base = wid * EPW
    pltpu.sync_copy(src_hbm.at[wid], idxs_v)
    pltpu.sync_copy(dst_hbm.at[wid], idxd_v)
    pltpu.sync_copy(cpx_hbm, cpx)
    pltpu.sync_copy(cpy_hbm, cpy)
    pltpu.sync_copy(cpz_hbm, cpz)

    bufa = (bufa0, bufa1)
    bufb = (bufb0, bufb1)
    bufxd = (bufxd0, bufxd1)
    gsem = (gsem0, gsem1)
    wsem = (wsem0, wsem1)

    zero16 = jnp.zeros((16,), jnp.float32)

    def zrow(r, carry):
        bufxd0[r] = zero16
        bufxd1[r] = zero16
        return carry

    lax.fori_loop(0, CH, zrow, 0)

    lanes = lax.iota(jnp.int32, 16)

    def fire_gather(j, sb):
        pltpu.async_copy(a_hbm.at[idxs_v.at[j]], bufa[sb], gsem[sb])
        pltpu.async_copy(b_hbm.at[idxd_v.at[j]], bufb[sb], gsem[sb])

    def wait_gather(j, sb):
        pltpu.make_async_copy(a_hbm.at[idxs_v.at[j]], bufa[sb],
                              gsem[sb]).wait()
        pltpu.make_async_copy(b_hbm.at[idxd_v.at[j]], bufb[sb],
                              gsem[sb]).wait()

    def fire_writes(j, sb):
        off = base + j * CH
        pltpu.async_copy(bufa[sb], g_hbm.at[pl.ds(off, CH)], wsem[sb])
        pltpu.async_copy(bufxd[sb], xd_hbm.at[pl.ds(off, CH)], wsem[sb])

    def wait_writes(sb):
        pltpu.make_async_copy(bufa[sb], g_hbm.at[pl.ds(0, CH)],
                              wsem[sb]).wait()
        pltpu.make_async_copy(bufxd[sb], xd_hbm.at[pl.ds(0, CH)],
                              wsem[sb]).wait()

    def compute(j, sb):
        bxd = bufxd[sb]
        ba = bufa[sb]
        bb = bufb[sb]

        def grp(k, carry2):
            sidx = idxs_v[j, pl.ds(k * 16, 16)]
            didx = idxd_v[j, pl.ds(k * 16, 16)]
            dx = plsc.load_gather(cpx, [sidx]) - plsc.load_gather(cpx, [didx])
            dy = plsc.load_gather(cpy, [sidx]) - plsc.load_gather(cpy, [didx])
            dz = plsc.load_gather(cpz, [sidx]) - plsc.load_gather(cpz, [didx])
            rad = dx * dx + dy * dy + dz * dz
            ridx = lanes + k * 16
            plsc.store_scatter(bxd, [ridx, jnp.zeros((16,), jnp.int32)], dx)
            plsc.store_scatter(bxd, [ridx, jnp.full((16,), 1, jnp.int32)], dy)
            plsc.store_scatter(bxd, [ridx, jnp.full((16,), 2, jnp.int32)], dz)
            plsc.store_scatter(bxd, [ridx, jnp.full((16,), 3, jnp.int32)], rad)
            return carry2

        lax.fori_loop(0, CH // 16, grp, 0)

        def row(r, carry2):
            for kk in range(HID // 16):
                sl = pl.ds(kk * 16, 16)
                ba[r, sl] = ba[r, sl] + bb[r, sl]
            return carry2

        lax.fori_loop(0, CH, row, 0)

    def do_chunk(j, sb, ob):
        @pl.when(j + 1 < NCHUNK)
        def _prefetch():
            @pl.when(j >= 1)
            def _drain():
                wait_writes(ob)

            fire_gather(j + 1, ob)

        wait_gather(j, sb)
        compute(j, sb)
        fire_writes(j, sb)

    fire_gather(0, 0)

    def pair(jj, carry):
        j0 = 2 * jj
        do_chunk(j0, 0, 1)

        @pl.when(j0 + 1 < NCHUNK)
        def _odd():
            do_chunk(j0 + 1, 1, 0)

        return carry

    lax.fori_loop(0, (NCHUNK + 1) // 2, pair, 0)
    wait_writes(0)
    wait_writes(1)


def _edge_body(g_ref, xd_ref, ef_ref, wef_ref, wr_ref, we2_ref, be2_ref,
               wc1_ref, bc1_ref, wc2_ref, msgh_ref, msgx_ref):
    g = g_ref[...]
    xd = xd_ref[...]
    radial = xd[:, 3:4]
    pre = (
        g
        + radial * wr_ref[...]
        + jnp.dot(ef_ref[...], wef_ref[...], preferred_element_type=jnp.float32)
    )
    m1 = _silu(pre)
    mh = _silu(
        jnp.dot(m1, we2_ref[...], preferred_element_type=jnp.float32)
        + be2_ref[...]
    )
    t = _silu(
        jnp.dot(mh, wc1_ref[...], preferred_element_type=jnp.float32)
        + bc1_ref[...]
    )
    coef = jnp.sum(t * wc2_ref[...], axis=1, keepdims=True)
    inv = 1.0 / (jnp.sqrt(radial) + 1e-30)
    lane = lax.broadcasted_iota(jnp.int32, (1, 16), 1)
    mask3 = jnp.where(lane < 3, 1.0, 0.0)
    deg1 = jnp.where(lane == 3, 1.0, 0.0)
    msgh_ref[...] = mh
    msgx_ref[...] = coef * inv * xd * mask3 + deg1


def _edge_mlp(g, xd, edge_feat, wef, wr, we2, be2, wc1, bc1, wc2_row):
    tile = 4000
    grid = E // tile
    return pl.pallas_call(
        _edge_body,
        grid=(grid,),
        in_specs=[
            pl.BlockSpec((tile, HID), lambda i: (i, 0)),
            pl.BlockSpec((tile, 16), lambda i: (i, 0)),
            pl.BlockSpec((tile, EF), lambda i: (i, 0)),
            pl.BlockSpec((EF, HID), lambda i: (0, 0)),
            pl.BlockSpec((1, HID), lambda i: (0, 0)),
            pl.BlockSpec((HID, HID), lambda i: (0, 0)),
            pl.BlockSpec((1, HID), lambda i: (0, 0)),
            pl.BlockSpec((HID, HID), lambda i: (0, 0)),
            pl.BlockSpec((1, HID), lambda i: (0, 0)),
            pl.BlockSpec((1, HID), lambda i: (0, 0)),
        ],
        out_specs=[
            pl.BlockSpec((tile, HID), lambda i: (i, 0)),
            pl.BlockSpec((tile, 16), lambda i: (i, 0)),
        ],
        out_shape=[
            jax.ShapeDtypeStruct((E, HID), jnp.float32),
            jax.ShapeDtypeStruct((E, 16), jnp.float32),
        ],
    )(g, xd, edge_feat, wef, wr, we2, be2, wc1, bc1, wc2_row)


HROWS = 10000


@functools.partial(
    pl.kernel,
    out_type=jax.ShapeDtypeStruct((NC, N, HID), jnp.float32),
    mesh=_MESH,
    scratch_types=[
        pltpu.VMEM((NCHUNK, CH), jnp.int32),
        pltpu.VMEM((CH, HID), jnp.float32),
        pltpu.VMEM_SHARED((HROWS, HID), jnp.float32),
        pltpu.SemaphoreType.DMA,
    ],
    compiler_params=pltpu.CompilerParams(needs_layout_passes=False),
)
def _scatter_h(msgh_hbm, dst_hbm, zh_hbm, hpart, idx_v, bufh, hacc, sem):
    cid = lax.axis_index("c")
    sid = lax.axis_index("s")
    wid = sid * NC + cid
    base = wid * EPW

    @pl.when(sid < 10)
    def _zero_main():
        pltpu.sync_copy(zh_hbm.at[pl.ds(sid * HZR, HZR)],
                        hacc.at[pl.ds(sid * HZR, HZR)])

    pltpu.sync_copy(dst_hbm.at[wid], idx_v)
    plsc.subcore_barrier()

    def chunk(j, carry):
        off = base + j * CH
        pltpu.async_copy(msgh_hbm.at[pl.ds(off, CH)], bufh, sem).wait()
        pltpu.sync_copy(bufh, hacc.at[idx_v.at[j]], add=True)
        return carry

    lax.fori_loop(0, NCHUNK, chunk, 0)
    plsc.subcore_barrier()

    @pl.when(sid < 10)
    def _copy_out():
        pltpu.sync_copy(hacc.at[pl.ds(sid * HZR, HZR)],
                        hpart.at[cid, pl.ds(sid * HZR, HZR)])


@functools.partial(
    pl.kernel,
    out_type=jax.ShapeDtypeStruct((NC, N, HID), jnp.float32),
    mesh=_MESH,
    scratch_types=[
        pltpu.VMEM((NCHUNK, CH), jnp.int32),
        pltpu.VMEM((CH, 16), jnp.float32),
        pltpu.VMEM((CH, HID), jnp.float32),
        pltpu.VMEM_SHARED((HROWS, HID), jnp.float32),
        pltpu.SemaphoreType.DMA,
    ],
    compiler_params=pltpu.CompilerParams(needs_layout_passes=False),
)
def _scatter_x(msgx_hbm, dst_hbm, zh_hbm, xpart, idx_v, bufx, bufx128, xacc,
               sem):
    cid = lax.axis_index("c")
    sid = lax.axis_index("s")
    wid = sid * NC + cid
    base = wid * EPW

    @pl.when(sid < 10)
    def _zero_main():
        pltpu.sync_copy(zh_hbm.at[pl.ds(sid * HZR, HZR)],
                        xacc.at[pl.ds(sid * HZR, HZR)])

    pltpu.sync_copy(dst_hbm.at[wid], idx_v)
    zero16 = jnp.zeros((16,), jnp.float32)

    def zrow(r, carry):
        for kk in range(1, HID // 16):
            bufx128[r, pl.ds(kk * 16, 16)] = zero16
        return carry

    lax.fori_loop(0, CH, zrow, 0)
    plsc.subcore_barrier()

    def chunk(j, carry):
        off = base + j * CH
        pltpu.async_copy(msgx_hbm.at[pl.ds(off, CH)], bufx, sem).wait()

        def row(r, carry2):
            bufx128[r, pl.ds(0, 16)] = bufx[r]
            return carry2

        lax.fori_loop(0, CH, row, 0)
        pltpu.sync_copy(bufx128, xacc.at[idx_v.at[j]], add=True)
        return carry

    lax.fori_loop(0, NCHUNK, chunk, 0)
    plsc.subcore_barrier()

    @pl.when(sid < 10)
    def _copy_out():
        pltpu.sync_copy(xacc.at[pl.ds(sid * HZR, HZR)],
                        xpart.at[cid, pl.ds(sid * HZR, HZR)])


def _node_body(nf_ref, cp_ref, h0_ref, h1_ref, x0_ref, x1_ref,
               wn1a_ref, wn1b_ref, bn1_ref, wn2_ref, bn2_ref,
               gamma_ref, beta_ref, h_ref, xp_ref):
    nf = nf_ref[...]
    hn = h0_ref[...] + h1_ref[...]
    xs = x0_ref[...][:, :16] + x1_ref[...][:, :16]
    deg = jnp.maximum(xs[:, 3:4], 1.0)
    lane = lax.broadcasted_iota(jnp.int32, (1, 16), 1)
    mask3 = jnp.where(lane < 3, 1.0, 0.0)
    h1v = _silu(
        jnp.dot(nf, wn1a_ref[...], preferred_element_type=jnp.float32)
        + jnp.dot(hn, wn1b_ref[...], preferred_element_type=jnp.float32)
        + bn1_ref[...]
    )
    h2 = (
        jnp.dot(h1v, wn2_ref[...], preferred_element_type=jnp.float32)
        + bn2_ref[...]
    )
    mean = jnp.mean(h2, axis=0, keepdims=True)
    var = jnp.mean(h2 * h2, axis=0, keepdims=True) - mean * mean
    h_ref[...] = (
        (h2 - mean) / jnp.sqrt(var + 1e-5) * gamma_ref[...] + beta_ref[...]
    )
    xp_ref[...] = cp_ref[...] + xs * mask3 / deg


def _node_update(node_feat, cp, h0, h1, x0, x1, wn1a, wn1b, bn1, wn2, bn2,
                 gamma, beta):
    return pl.pallas_call(
        _node_body,
        out_shape=[
            jax.ShapeDtypeStruct((N, OUT), jnp.float32),
            jax.ShapeDtypeStruct((N, 16), jnp.float32),
        ],
    )(node_feat, cp, h0, h1, x0, x1, wn1a, wn1b, bn1, wn2, bn2, gamma, beta)


def kernel(node_feat, coord_feat, edge_index, edge_feat, W_e1, b_e1, W_e2,
           b_e2, W_n1, b_n1, W_n2, b_n2, W_c1, b_c1, W_c2, bn_gamma, bn_beta):
    f32 = jnp.float32
    w1a = W_e1[:D]
    w1b = W_e1[D:2 * D]
    wr = W_e1[2 * D:2 * D + 1]
    wef = W_e1[2 * D + 1:]
    be1 = b_e1.reshape(1, HID)
    be2 = b_e2.reshape(1, HID)
    bc1 = b_c1.reshape(1, HID)
    bn1 = b_n1.reshape(1, HID)
    bn2 = b_n2.reshape(1, OUT)
    wc2_row = W_c2.reshape(1, HID)
    gamma = bn_gamma.reshape(1, OUT)
    beta = bn_beta.reshape(1, OUT)
    wn1a = W_n1[:D]
    wn1b = W_n1[D:]

    cp = jnp.pad(coord_feat, ((0, 0), (0, 13)))
    cpx_t = jnp.asarray(coord_feat[:, 0], f32)
    cpy_t = jnp.asarray(coord_feat[:, 1], f32)
    cpz_t = jnp.asarray(coord_feat[:, 2], f32)
    src3 = edge_index[0].reshape(NW, NCHUNK, CH)
    dst3 = edge_index[1].reshape(NW, NCHUNK, CH)

    a, b = _node_precompute(node_feat, w1a, w1b, be1)
    g, xd = _edge_gather(a, b, cpx_t, cpy_t, cpz_t, src3, dst3)
    msgh, msgx = _edge_mlp(g, xd, edge_feat, wef, wr, W_e2, be2, W_c1, bc1,
                           wc2_row)
    zh = jnp.zeros((HROWS, HID), f32)
    hpart = _scatter_h(msgh, dst3, zh)
    xpart = _scatter_x(msgx, dst3, zh)
    h, xp = _node_update(node_feat, cp, hpart[0], hpart[1], xpart[0],
                         xpart[1], wn1a, wn1b, bn1, W_n2, bn2, gamma, beta)
    return (h, xp[:, :3])


# speedup vs baseline: 1.3136x; 1.0212x over previous
import functools

import jax
import jax.numpy as jnp
from jax import lax
from jax.experimental import pallas as pl
from jax.experimental.pallas import tpu as pltpu
from jax.experimental.pallas import tpu_sc as plsc

N = 10000
E = 320000
D = 128
HID = 128
OUT = 128
EF = 16

NC = 2
NS = 16
NW = NC * NS
EPW = E // NW
CH = 80
NCHUNK = EPW // CH
HZR = 1000


def _silu(x):
    return x * jax.nn.sigmoid(x)


def _pre_body(nf_ref, w1a_ref, w1b_ref, be1_ref, a_ref, b_ref):
    x = nf_ref[...]
    a_ref[...] = (
        jnp.dot(x, w1a_ref[...], preferred_element_type=jnp.float32)
        + be1_ref[...]
    )
    b_ref[...] = jnp.dot(x, w1b_ref[...], preferred_element_type=jnp.float32)


def _node_precompute(node_feat, w1a, w1b, be1):
    tile = 2000
    grid = N // tile
    return pl.pallas_call(
        _pre_body,
        grid=(grid,),
        in_specs=[
            pl.BlockSpec((tile, D), lambda i: (i, 0)),
            pl.BlockSpec((D, HID), lambda i: (0, 0)),
            pl.BlockSpec((D, HID), lambda i: (0, 0)),
            pl.BlockSpec((1, HID), lambda i: (0, 0)),
        ],
        out_specs=[
            pl.BlockSpec((tile, HID), lambda i: (i, 0)),
            pl.BlockSpec((tile, HID), lambda i: (i, 0)),
        ],
        out_shape=[
            jax.ShapeDtypeStruct((N, HID), jnp.float32),
            jax.ShapeDtypeStruct((N, HID), jnp.float32),
        ],
    )(node_feat, w1a, w1b, be1)


_MESH = plsc.VectorSubcoreMesh(core_axis_name="c", subcore_axis_name="s")


@functools.partial(
    pl.kernel,
    out_type=(
        jax.ShapeDtypeStruct((E, HID), jnp.float32),
        jax.ShapeDtypeStruct((E, 16), jnp.float32),
    ),
    mesh=_MESH,
    scratch_types=[
        pltpu.VMEM((NCHUNK, CH), jnp.int32),
        pltpu.VMEM((NCHUNK, CH), jnp.int32),
        pltpu.VMEM((CH, HID), jnp.float32),
        pltpu.VMEM((CH, HID), jnp.float32),
        pltpu.VMEM((CH, HID), jnp.float32),
        pltpu.VMEM((CH, HID), jnp.float32),
        pltpu.VMEM((N,), jnp.float32),
        pltpu.VMEM((N,), jnp.float32),
        pltpu.VMEM((N,), jnp.float32),
        pltpu.VMEM((CH, 16), jnp.float32),
        pltpu.VMEM((CH, 16), jnp.float32),
        pltpu.SemaphoreType.DMA,
        pltpu.SemaphoreType.DMA,
        pltpu.SemaphoreType.DMA,
        pltpu.SemaphoreType.DMA,
    ],
    compiler_params=pltpu.CompilerParams(needs_layout_passes=False),
)
def _edge_gather(a_hbm, b_hbm, cpx_hbm, cpy_hbm, cpz_hbm, src_hbm, dst_hbm,
                 g_hbm, xd_hbm,
                 idxs_v, idxd_v, bufa0, bufa1, bufb0, bufb1, cpx, cpy, cpz,
                 bufxd0, bufxd1, gsem0, gsem1, wsem0, wsem1):
    wid = lax.axis_index("s") * NC + lax.axis_index("c")
    base = wid * EPW
    pltpu.sync_copy(src_hbm.at[wid], idxs_v)
    pltpu.sync_copy(dst_hbm.at[wid], idxd_v)
    pltpu.sync_copy(cpx_hbm, cpx)
    pltpu.sync_copy(cpy_hbm, cpy)
    pltpu.sync_copy(cpz_hbm, cpz)

    bufa = (bufa0, bufa1)
    bufb = (bufb0, bufb1)
    bufxd = (bufxd0, bufxd1)
    gsem = (gsem0, gsem1)
    wsem = (wsem0, wsem1)

    zero16 = jnp.zeros((16,), jnp.float32)

    def zrow(r, carry):
        bufxd0[r] = zero16
        bufxd1[r] = zero16
        return carry

    lax.fori_loop(0, CH, zrow, 0)

    lanes = lax.iota(jnp.int32, 16)

    def fire_gather(j, sb):
        pltpu.async_copy(a_hbm.at[idxs_v.at[j]], bufa[sb], gsem[sb])
        pltpu.async_copy(b_hbm.at[idxd_v.at[j]], bufb[sb], gsem[sb])

    def wait_gather(j, sb):
        pltpu.make_async_copy(a_hbm.at[idxs_v.at[j]], bufa[sb],
                              gsem[sb]).wait()
        pltpu.make_async_copy(b_hbm.at[idxd_v.at[j]], bufb[sb],
                              gsem[sb]).wait()

    def fire_writes(j, sb):
        off = base + j * CH
        pltpu.async_copy(bufa[sb], g_hbm.at[pl.ds(off, CH)], wsem[sb])
        pltpu.async_copy(bufxd[sb], xd_hbm.at[pl.ds(off, CH)], wsem[sb])

    def wait_writes(sb):
        pltpu.make_async_copy(bufa[sb], g_hbm.at[pl.ds(0, CH)],
                              wsem[sb]).wait()
        pltpu.make_async_copy(bufxd[sb], xd_hbm.at[pl.ds(0, CH)],
                              wsem[sb]).wait()

    def compute(j, sb):
        bxd = bufxd[sb]
        ba = bufa[sb]
        bb = bufb[sb]

        def grp(k, carry2):
            sidx = idxs_v[j, pl.ds(k * 16, 16)]
            didx = idxd_v[j, pl.ds(k * 16, 16)]
            dx = plsc.load_gather(cpx, [sidx]) - plsc.load_gather(cpx, [didx])
            dy = plsc.load_gather(cpy, [sidx]) - plsc.load_gather(cpy, [didx])
            dz = plsc.load_gather(cpz, [sidx]) - plsc.load_gather(cpz, [didx])
            rad = dx * dx + dy * dy + dz * dz
            ridx = lanes + k * 16
            plsc.store_scatter(bxd, [ridx, jnp.zeros((16,), jnp.int32)], dx)
            plsc.store_scatter(bxd, [ridx, jnp.full((16,), 1, jnp.int32)], dy)
            plsc.store_scatter(bxd, [ridx, jnp.full((16,), 2, jnp.int32)], dz)
            plsc.store_scatter(bxd, [ridx, jnp.full((16,), 3, jnp.int32)], rad)
            return carry2

        lax.fori_loop(0, CH // 16, grp, 0)

        def row(r, carry2):
            for kk in range(HID // 16):
                sl = pl.ds(kk * 16, 16)
                ba[r, sl] = ba[r, sl] + bb[r, sl]
            return carry2

        lax.fori_loop(0, CH, row, 0)

    def do_chunk(j, sb, ob):
        @pl.when(j + 1 < NCHUNK)
        def _prefetch():
            @pl.when(j >= 1)
            def _drain():
                wait_writes(ob)

            fire_gather(j + 1, ob)

        wait_gather(j, sb)
        compute(j, sb)
        fire_writes(j, sb)

    fire_gather(0, 0)

    def pair(jj, carry):
        j0 = 2 * jj
        do_chunk(j0, 0, 1)

        @pl.when(j0 + 1 < NCHUNK)
        def _odd():
            do_chunk(j0 + 1, 1, 0)

        return carry

    lax.fori_loop(0, (NCHUNK + 1) // 2, pair, 0)
    wait_writes(0)
    wait_writes(1)


def _edge_body(g_ref, xd_ref, ef_ref, wef_ref, wr_ref, we2_ref, be2_ref,
               wc1_ref, bc1_ref, wc2_ref, msgh_ref, msgx_ref):
    g = g_ref[...]
    xd = xd_ref[...]
    radial = xd[:, 3:4]
    pre = (
        g
        + radial * wr_ref[...]
        + jnp.dot(ef_ref[...], wef_ref[...], preferred_element_type=jnp.float32)
    )
    m1 = _silu(pre)
    mh = _silu(
        jnp.dot(m1, we2_ref[...], preferred_element_type=jnp.float32)
        + be2_ref[...]
    )
    t = _silu(
        jnp.dot(mh, wc1_ref[...], preferred_element_type=jnp.float32)
        + bc1_ref[...]
    )
    coef = jnp.sum(t * wc2_ref[...], axis=1, keepdims=True)
    inv = 1.0 / (jnp.sqrt(radial) + 1e-30)
    lane = lax.broadcasted_iota(jnp.int32, (1, 16), 1)
    mask3 = jnp.where(lane < 3, 1.0, 0.0)
    deg1 = jnp.where(lane == 3, 1.0, 0.0)
    msgh_ref[...] = mh
    msgx_ref[...] = coef * inv * xd * mask3 + deg1


def _edge_mlp(g, xd, edge_feat, wef, wr, we2, be2, wc1, bc1, wc2_row):
    tile = 8000
    grid = E // tile
    return pl.pallas_call(
        _edge_body,
        grid=(grid,),
        in_specs=[
            pl.BlockSpec((tile, HID), lambda i: (i, 0)),
            pl.BlockSpec((tile, 16), lambda i: (i, 0)),
            pl.BlockSpec((tile, EF), lambda i: (i, 0)),
            pl.BlockSpec((EF, HID), lambda i: (0, 0)),
            pl.BlockSpec((1, HID), lambda i: (0, 0)),
            pl.BlockSpec((HID, HID), lambda i: (0, 0)),
            pl.BlockSpec((1, HID), lambda i: (0, 0)),
            pl.BlockSpec((HID, HID), lambda i: (0, 0)),
            pl.BlockSpec((1, HID), lambda i: (0, 0)),
            pl.BlockSpec((1, HID), lambda i: (0, 0)),
        ],
        out_specs=[
            pl.BlockSpec((tile, HID), lambda i: (i, 0)),
            pl.BlockSpec((tile, 16), lambda i: (i, 0)),
        ],
        out_shape=[
            jax.ShapeDtypeStruct((E, HID), jnp.float32),
            jax.ShapeDtypeStruct((E, 16), jnp.float32),
        ],
    )(g, xd, edge_feat, wef, wr, we2, be2, wc1, bc1, wc2_row)


HROWS = 10000


@functools.partial(
    pl.kernel,
    out_type=jax.ShapeDtypeStruct((NC, N, HID), jnp.float32),
    mesh=_MESH,
    scratch_types=[
        pltpu.VMEM((NCHUNK, CH), jnp.int32),
        pltpu.VMEM((CH, HID), jnp.float32),
        pltpu.VMEM_SHARED((HROWS, HID), jnp.float32),
        pltpu.SemaphoreType.DMA,
    ],
    compiler_params=pltpu.CompilerParams(needs_layout_passes=False),
)
def _scatter_h(msgh_hbm, dst_hbm, zh_hbm, hpart, idx_v, bufh, hacc, sem):
    cid = lax.axis_index("c")
    sid = lax.axis_index("s")
    wid = sid * NC + cid
    base = wid * EPW

    @pl.when(sid < 10)
    def _zero_main():
        pltpu.sync_copy(zh_hbm.at[pl.ds(sid * HZR, HZR)],
                        hacc.at[pl.ds(sid * HZR, HZR)])

    pltpu.sync_copy(dst_hbm.at[wid], idx_v)
    plsc.subcore_barrier()

    def chunk(j, carry):
        off = base + j * CH
        pltpu.async_copy(msgh_hbm.at[pl.ds(off, CH)], bufh, sem).wait()
        pltpu.sync_copy(bufh, hacc.at[idx_v.at[j]], add=True)
        return carry

    lax.fori_loop(0, NCHUNK, chunk, 0)
    plsc.subcore_barrier()

    @pl.when(sid < 10)
    def _copy_out():
        pltpu.sync_copy(hacc.at[pl.ds(sid * HZR, HZR)],
                        hpart.at[cid, pl.ds(sid * HZR, HZR)])


@functools.partial(
    pl.kernel,
    out_type=jax.ShapeDtypeStruct((NC, N, HID), jnp.float32),
    mesh=_MESH,
    scratch_types=[
        pltpu.VMEM((NCHUNK, CH), jnp.int32),
        pltpu.VMEM((CH, 16), jnp.float32),
        pltpu.VMEM((CH, HID), jnp.float32),
        pltpu.VMEM_SHARED((HROWS, HID), jnp.float32),
        pltpu.SemaphoreType.DMA,
    ],
    compiler_params=pltpu.CompilerParams(needs_layout_passes=False),
)
def _scatter_x(msgx_hbm, dst_hbm, zh_hbm, xpart, idx_v, bufx, bufx128, xacc,
               sem):
    cid = lax.axis_index("c")
    sid = lax.axis_index("s")
    wid = sid * NC + cid
    base = wid * EPW

    @pl.when(sid < 10)
    def _zero_main():
        pltpu.sync_copy(zh_hbm.at[pl.ds(sid * HZR, HZR)],
                        xacc.at[pl.ds(sid * HZR, HZR)])

    pltpu.sync_copy(dst_hbm.at[wid], idx_v)
    zero16 = jnp.zeros((16,), jnp.float32)

    def zrow(r, carry):
        for kk in range(1, HID // 16):
            bufx128[r, pl.ds(kk * 16, 16)] = zero16
        return carry

    lax.fori_loop(0, CH, zrow, 0)
    plsc.subcore_barrier()

    def chunk(j, carry):
        off = base + j * CH
        pltpu.async_copy(msgx_hbm.at[pl.ds(off, CH)], bufx, sem).wait()

        def row(r, carry2):
            bufx128[r, pl.ds(0, 16)] = bufx[r]
            return carry2

        lax.fori_loop(0, CH, row, 0)
        pltpu.sync_copy(bufx128, xacc.at[idx_v.at[j]], add=True)
        return carry

    lax.fori_loop(0, NCHUNK, chunk, 0)
    plsc.subcore_barrier()

    @pl.when(sid < 10)
    def _copy_out():
        pltpu.sync_copy(xacc.at[pl.ds(sid * HZR, HZR)],
                        xpart.at[cid, pl.ds(sid * HZR, HZR)])


def _node_body(nf_ref, cp_ref, h0_ref, h1_ref, x0_ref, x1_ref,
               wn1a_ref, wn1b_ref, bn1_ref, wn2_ref, bn2_ref,
               gamma_ref, beta_ref, h_ref, xp_ref):
    nf = nf_ref[...]
    hn = h0_ref[...] + h1_ref[...]
    xs = x0_ref[...][:, :16] + x1_ref[...][:, :16]
    deg = jnp.maximum(xs[:, 3:4], 1.0)
    lane = lax.broadcasted_iota(jnp.int32, (1, 16), 1)
    mask3 = jnp.where(lane < 3, 1.0, 0.0)
    h1v = _silu(
        jnp.dot(nf, wn1a_ref[...], preferred_element_type=jnp.float32)
        + jnp.dot(hn, wn1b_ref[...], preferred_element_type=jnp.float32)
        + bn1_ref[...]
    )
    h2 = (
        jnp.dot(h1v, wn2_ref[...], preferred_element_type=jnp.float32)
        + bn2_ref[...]
    )
    mean = jnp.mean(h2, axis=0, keepdims=True)
    var = jnp.mean(h2 * h2, axis=0, keepdims=True) - mean * mean
    h_ref[...] = (
        (h2 - mean) / jnp.sqrt(var + 1e-5) * gamma_ref[...] + beta_ref[...]
    )
    xp_ref[...] = cp_ref[...] + xs * mask3 / deg


def _node_update(node_feat, cp, h0, h1, x0, x1, wn1a, wn1b, bn1, wn2, bn2,
                 gamma, beta):
    return pl.pallas_call(
        _node_body,
        out_shape=[
            jax.ShapeDtypeStruct((N, OUT), jnp.float32),
            jax.ShapeDtypeStruct((N, 16), jnp.float32),
        ],
    )(node_feat, cp, h0, h1, x0, x1, wn1a, wn1b, bn1, wn2, bn2, gamma, beta)


def kernel(node_feat, coord_feat, edge_index, edge_feat, W_e1, b_e1, W_e2,
           b_e2, W_n1, b_n1, W_n2, b_n2, W_c1, b_c1, W_c2, bn_gamma, bn_beta):
    f32 = jnp.float32
    w1a = W_e1[:D]
    w1b = W_e1[D:2 * D]
    wr = W_e1[2 * D:2 * D + 1]
    wef = W_e1[2 * D + 1:]
    be1 = b_e1.reshape(1, HID)
    be2 = b_e2.reshape(1, HID)
    bc1 = b_c1.reshape(1, HID)
    bn1 = b_n1.reshape(1, HID)
    bn2 = b_n2.reshape(1, OUT)
    wc2_row = W_c2.reshape(1, HID)
    gamma = bn_gamma.reshape(1, OUT)
    beta = bn_beta.reshape(1, OUT)
    wn1a = W_n1[:D]
    wn1b = W_n1[D:]

    cp = jnp.pad(coord_feat, ((0, 0), (0, 13)))
    cpx_t = jnp.asarray(coord_feat[:, 0], f32)
    cpy_t = jnp.asarray(coord_feat[:, 1], f32)
    cpz_t = jnp.asarray(coord_feat[:, 2], f32)
    src3 = edge_index[0].reshape(NW, NCHUNK, CH)
    dst3 = edge_index[1].reshape(NW, NCHUNK, CH)

    a, b = _node_precompute(node_feat, w1a, w1b, be1)
    g, xd = _edge_gather(a, b, cpx_t, cpy_t, cpz_t, src3, dst3)
    msgh, msgx = _edge_mlp(g, xd, edge_feat, wef, wr, W_e2, be2, W_c1, bc1,
                           wc2_row)
    zh = jnp.zeros((HROWS, HID), f32)
    hpart = _scatter_h(msgh, dst3, zh)
    xpart = _scatter_x(msgx, dst3, zh)
    h, xp = _node_update(node_feat, cp, hpart[0], hpart[1], xpart[0],
                         xpart[1], wn1a, wn1b, bn1, W_n2, bn2, gamma, beta)
    return (h, xp[:, :3])
